# baseline (speedup 1.0000x reference)
import functools

import jax
import jax.numpy as jnp
from jax import lax
from jax.experimental import pallas as pl
from jax.experimental.pallas import tpu as pltpu
from jax.experimental.pallas import tpu_sc as plsc

N = 10000
E = 320000
D = 128

NC = 2
NS = 16
NW = NC * NS
EPW = E // NW
CHS = 128
NFC = EPW // CHS
TL = EPW - NFC * CHS
NP = 10240
RPS = NP // NS
ZR = 128
DW = 16
SCH = 80
SNC = EPW // SCH


def _deg_body(dst_hbm, out_hbm, ones_v, zbuf_v, idx_v, acc_sh):
    c = lax.axis_index("c")
    s = lax.axis_index("s")
    wid = c * NS + s

    @pl.loop(0, SCH)
    def _ofill(i):
        ones_v[i] = jnp.ones((DW,), jnp.float32)
        zbuf_v[i] = jnp.zeros((DW,), jnp.float32)

    @pl.loop(0, RPS // SCH)
    def _zero(k):
        pltpu.sync_copy(zbuf_v, acc_sh.at[pl.ds(s * RPS + k * SCH, SCH)])

    plsc.subcore_barrier()

    @pl.loop(0, SNC)
    def _edges(i):
        e0 = wid * EPW + i * SCH
        pltpu.sync_copy(dst_hbm.at[pl.ds(e0, SCH)], idx_v)
        pltpu.sync_copy(ones_v, acc_sh.at[idx_v], add=True)

    plsc.subcore_barrier()
    pltpu.sync_copy(
        acc_sh.at[pl.ds(s * RPS, RPS)], out_hbm.at[c, pl.ds(s * RPS, RPS)]
    )


@functools.cache
def _deg_call():
    return pl.kernel(
        _deg_body,
        out_type=jax.ShapeDtypeStruct((NC, NP, DW), jnp.float32),
        mesh=plsc.VectorSubcoreMesh(
            core_axis_name="c", subcore_axis_name="s",
            num_cores=NC, num_subcores=NS,
        ),
        scratch_types=[
            pltpu.VMEM((SCH, DW), jnp.float32),
            pltpu.VMEM((SCH, DW), jnp.float32),
            pltpu.VMEM((SCH,), jnp.int32),
            pltpu.VMEM_SHARED((NP, DW), jnp.float32),
        ],
    )


def _scat_body(hp_hbm, src_hbm, dst_hbm, out_hbm,
               idxs0_v, idxd0_v, idxs1_v, idxd1_v, rows0_v, rows1_v, acc_sh,
               sem0, sem1):
    c = lax.axis_index("c")
    s = lax.axis_index("s")
    wid = c * NS + s

    @pl.loop(0, SCH)
    def _zfill(i):
        for j in range(D // 16):
            rows0_v[i, pl.ds(j * 16, 16)] = jnp.zeros((16,), jnp.float32)

    @pl.loop(0, RPS // SCH)
    def _zero(k):
        pltpu.sync_copy(rows0_v, acc_sh.at[pl.ds(s * RPS + k * SCH, SCH)])

    plsc.subcore_barrier()

    def _load(i, idxs_v, idxd_v):
        e0 = wid * EPW + i * SCH
        pltpu.sync_copy(src_hbm.at[pl.ds(e0, SCH)], idxs_v)
        pltpu.sync_copy(dst_hbm.at[pl.ds(e0, SCH)], idxd_v)

    def _gather(idxs_v, buf, sem):
        pltpu.async_copy(hp_hbm.at[idxs_v], buf, sem)

    def _gwait(idxs_v, buf, sem):
        pltpu.make_async_copy(hp_hbm.at[idxs_v], buf, sem).wait()

    def _scatter(idxd_v, buf):
        pltpu.sync_copy(buf, acc_sh.at[idxd_v], add=True)

    _load(0, idxs0_v, idxd0_v)
    _gather(idxs0_v, rows0_v, sem0)

    @pl.loop(0, SNC - 1, step=2)
    def _edges(i):
        _load(i + 1, idxs1_v, idxd1_v)
        _gather(idxs1_v, rows1_v, sem1)
        _gwait(idxs0_v, rows0_v, sem0)
        _scatter(idxd0_v, rows0_v)
        _load(i + 2, idxs0_v, idxd0_v)
        _gather(idxs0_v, rows0_v, sem0)
        _gwait(idxs1_v, rows1_v, sem1)
        _scatter(idxd1_v, rows1_v)

    _gwait(idxs0_v, rows0_v, sem0)
    _scatter(idxd0_v, rows0_v)

    plsc.subcore_barrier()
    pltpu.sync_copy(
        acc_sh.at[pl.ds(s * RPS, RPS)], out_hbm.at[c, pl.ds(s * RPS, RPS)]
    )


@functools.cache
def _scat_call():
    return pl.kernel(
        _scat_body,
        out_type=jax.ShapeDtypeStruct((NC, NP, D), jnp.float32),
        mesh=plsc.VectorSubcoreMesh(
            core_axis_name="c", subcore_axis_name="s",
            num_cores=NC, num_subcores=NS,
        ),
        scratch_types=[
            pltpu.VMEM((SCH,), jnp.int32),
            pltpu.VMEM((SCH,), jnp.int32),
            pltpu.VMEM((SCH,), jnp.int32),
            pltpu.VMEM((SCH,), jnp.int32),
            pltpu.VMEM((SCH, D), jnp.float32),
            pltpu.VMEM((SCH, D), jnp.float32),
            pltpu.VMEM_SHARED((NP, D), jnp.float32),
            pltpu.SemaphoreType.DMA,
            pltpu.SemaphoreType.DMA,
        ],
    )


BR = 1000
NB = N // BR


def _k1_body(dp_ref, x_ref, w_ref, hp_ref, dinv_ref):
    deg = 1.0 + dp_ref[0, :, 0:1] + dp_ref[1, :, 0:1]
    dinv = lax.rsqrt(deg)
    h = jnp.dot(x_ref[...], w_ref[...], preferred_element_type=jnp.float32)
    hp_ref[...] = h * dinv
    dinv_ref[...] = dinv


_k1_call = pl.pallas_call(
    _k1_body,
    grid=(NB,),
    in_specs=[
        pl.BlockSpec((NC, BR, DW), lambda i: (0, i, 0)),
        pl.BlockSpec((BR, D), lambda i: (i, 0)),
        pl.BlockSpec((D, D), lambda i: (0, 0)),
    ],
    out_specs=[
        pl.BlockSpec((BR, D), lambda i: (i, 0)),
        pl.BlockSpec((BR, 1), lambda i: (i, 0)),
    ],
    out_shape=[
        jax.ShapeDtypeStruct((N, D), jnp.float32),
        jax.ShapeDtypeStruct((N, 1), jnp.float32),
    ],
)


def _acc_body(p_ref, hp_ref, dinv_ref, b_ref, y_ref, st_ref, acc):
    i = pl.program_id(0)

    @pl.when(i == 0)
    def _():
        acc[...] = jnp.zeros_like(acc)

    y = dinv_ref[...] * (p_ref[0] + p_ref[1] + hp_ref[...]) + b_ref[...]
    y_ref[...] = y
    acc[0:1, :] = acc[0:1, :] + jnp.sum(y, axis=0, keepdims=True)
    acc[1:2, :] = acc[1:2, :] + jnp.sum(y * y, axis=0, keepdims=True)
    st_ref[...] = acc[...]


_acc_call = pl.pallas_call(
    _acc_body,
    grid=(NB,),
    in_specs=[
        pl.BlockSpec((NC, BR, D), lambda i: (0, i, 0)),
        pl.BlockSpec((BR, D), lambda i: (i, 0)),
        pl.BlockSpec((BR, 1), lambda i: (i, 0)),
        pl.BlockSpec((1, D), lambda i: (0, 0)),
    ],
    out_specs=[
        pl.BlockSpec((BR, D), lambda i: (i, 0)),
        pl.BlockSpec((2, D), lambda i: (0, 0)),
    ],
    out_shape=[
        jax.ShapeDtypeStruct((N, D), jnp.float32),
        jax.ShapeDtypeStruct((2, D), jnp.float32),
    ],
    scratch_shapes=[pltpu.VMEM((2, D), jnp.float32)],
)


def _bn_prelu(y, st, g_ref, be_ref, a_ref):
    mu = st[0:1, :] * (1.0 / N)
    var = st[1:2, :] * (1.0 / N) - mu * mu
    z = (y - mu) * lax.rsqrt(var + 1e-5) * g_ref[...] + be_ref[...]
    return jnp.where(z > 0, z, a_ref[0, 0] * z)


def _mid_body(y_ref, st_ref, g_ref, be_ref, a_ref, dinv_ref, w_ref, out_ref):
    p = _bn_prelu(y_ref[...], st_ref[...], g_ref, be_ref, a_ref)
    h = jnp.dot(p, w_ref[...], preferred_element_type=jnp.float32)
    out_ref[...] = h * dinv_ref[...]


_mid_call = pl.pallas_call(
    _mid_body,
    grid=(NB,),
    in_specs=[
        pl.BlockSpec((BR, D), lambda i: (i, 0)),
        pl.BlockSpec((2, D), lambda i: (0, 0)),
        pl.BlockSpec((1, D), lambda i: (0, 0)),
        pl.BlockSpec((1, D), lambda i: (0, 0)),
        pl.BlockSpec((1, 1), lambda i: (0, 0)),
        pl.BlockSpec((BR, 1), lambda i: (i, 0)),
        pl.BlockSpec((D, D), lambda i: (0, 0)),
    ],
    out_specs=pl.BlockSpec((BR, D), lambda i: (i, 0)),
    out_shape=jax.ShapeDtypeStruct((N, D), jnp.float32),
)


def _fin_body(y_ref, st_ref, g_ref, be_ref, a_ref, out_ref):
    out_ref[...] = _bn_prelu(y_ref[...], st_ref[...], g_ref, be_ref, a_ref)


_fin_call = pl.pallas_call(
    _fin_body,
    grid=(NB,),
    in_specs=[
        pl.BlockSpec((BR, D), lambda i: (i, 0)),
        pl.BlockSpec((2, D), lambda i: (0, 0)),
        pl.BlockSpec((1, D), lambda i: (0, 0)),
        pl.BlockSpec((1, D), lambda i: (0, 0)),
        pl.BlockSpec((1, 1), lambda i: (0, 0)),
    ],
    out_specs=pl.BlockSpec((BR, D), lambda i: (i, 0)),
    out_shape=jax.ShapeDtypeStruct((N, D), jnp.float32),
)


def kernel(x, edge_index, W1, b1, g1, be1, a1, W2, b2, g2, be2, a2):
    src = edge_index[0]
    dst = edge_index[1]
    b1r, b2r = b1.reshape(1, D), b2.reshape(1, D)
    g1r, g2r = g1.reshape(1, D), g2.reshape(1, D)
    be1r, be2r = be1.reshape(1, D), be2.reshape(1, D)
    a1r, a2r = a1.reshape(1, 1), a2.reshape(1, 1)

    degp = _deg_call()(dst)
    hp1, dinv = _k1_call(degp, x, W1)
    p1 = _scat_call()(hp1, src, dst)
    y1, st1 = _acc_call(p1, hp1, dinv, b1r)
    hp2 = _mid_call(y1, st1, g1r, be1r, a1r, dinv, W2)
    p2 = _scat_call()(hp2, src, dst)
    y2, st2 = _acc_call(p2, hp2, dinv, b2r)
    return _fin_call(y2, st2, g2r, be2r, a2r)


# speedup vs baseline: 1.0039x; 1.0039x over previous
import functools

import jax
import jax.numpy as jnp
from jax import lax
from jax.experimental import pallas as pl
from jax.experimental.pallas import tpu as pltpu
from jax.experimental.pallas import tpu_sc as plsc

N = 10000
E = 320000
D = 128

NC = 2
NS = 16
NW = NC * NS
EPW = E // NW
CHS = 128
NFC = EPW // CHS
TL = EPW - NFC * CHS
NP = 10240
RPS = NP // NS
ZR = 128
DW = 16
SCH = 80
SNC = EPW // SCH


def _deg_body(dst_hbm, out_hbm, ones_v, zbuf_v, idx_v, acc_sh):
    c = lax.axis_index("c")
    s = lax.axis_index("s")
    wid = c * NS + s

    @pl.loop(0, SCH)
    def _ofill(i):
        ones_v[i] = jnp.ones((DW,), jnp.float32)
        zbuf_v[i] = jnp.zeros((DW,), jnp.float32)

    @pl.loop(0, RPS // SCH)
    def _zero(k):
        pltpu.sync_copy(zbuf_v, acc_sh.at[pl.ds(s * RPS + k * SCH, SCH)])

    plsc.subcore_barrier()

    @pl.loop(0, SNC)
    def _edges(i):
        e0 = wid * EPW + i * SCH
        pltpu.sync_copy(dst_hbm.at[pl.ds(e0, SCH)], idx_v)
        pltpu.sync_copy(ones_v, acc_sh.at[idx_v], add=True)

    plsc.subcore_barrier()
    pltpu.sync_copy(
        acc_sh.at[pl.ds(s * RPS, RPS)], out_hbm.at[c, pl.ds(s * RPS, RPS)]
    )


@functools.cache
def _deg_call():
    return pl.kernel(
        _deg_body,
        out_type=jax.ShapeDtypeStruct((NC, NP, DW), jnp.float32),
        mesh=plsc.VectorSubcoreMesh(
            core_axis_name="c", subcore_axis_name="s",
            num_cores=NC, num_subcores=NS,
        ),
        scratch_types=[
            pltpu.VMEM((SCH, DW), jnp.float32),
            pltpu.VMEM((SCH, DW), jnp.float32),
            pltpu.VMEM((SCH,), jnp.int32),
            pltpu.VMEM_SHARED((NP, DW), jnp.float32),
        ],
    )


def _scat_body(hp_hbm, src_hbm, dst_hbm, out_hbm,
               idxs0_v, idxd0_v, idxs1_v, idxd1_v, idxs2_v, idxd2_v,
               rows0_v, rows1_v, rows2_v, acc_sh, sem0, sem1, sem2):
    c = lax.axis_index("c")
    s = lax.axis_index("s")
    wid = c * NS + s

    idxs = (idxs0_v, idxs1_v, idxs2_v)
    idxd = (idxd0_v, idxd1_v, idxd2_v)
    rows = (rows0_v, rows1_v, rows2_v)
    sems = (sem0, sem1, sem2)

    @pl.loop(0, SCH)
    def _zfill(i):
        for j in range(D // 16):
            rows0_v[i, pl.ds(j * 16, 16)] = jnp.zeros((16,), jnp.float32)

    @pl.loop(0, RPS // SCH)
    def _zero(k):
        pltpu.sync_copy(rows0_v, acc_sh.at[pl.ds(s * RPS + k * SCH, SCH)])

    plsc.subcore_barrier()

    def _issue(i, k):
        e0 = wid * EPW + i * SCH
        pltpu.sync_copy(src_hbm.at[pl.ds(e0, SCH)], idxs[k])
        pltpu.sync_copy(dst_hbm.at[pl.ds(e0, SCH)], idxd[k])
        pltpu.async_copy(hp_hbm.at[idxs[k]], rows[k], sems[k])

    def _drain(k):
        pltpu.make_async_copy(hp_hbm.at[idxs[k]], rows[k], sems[k]).wait()
        pltpu.sync_copy(rows[k], acc_sh.at[idxd[k]], add=True)

    for k in range(3):
        _issue(k, k)

    @pl.loop(0, SNC - 5, step=3)
    def _edges(i):
        for k in range(3):
            _drain(k)
            _issue(i + 3 + k, k)

    _drain(0)
    _issue(SNC - 2, 0)
    _drain(1)
    _issue(SNC - 1, 1)
    _drain(2)
    _drain(0)
    _drain(1)

    plsc.subcore_barrier()
    pltpu.sync_copy(
        acc_sh.at[pl.ds(s * RPS, RPS)], out_hbm.at[c, pl.ds(s * RPS, RPS)]
    )


@functools.cache
def _scat_call():
    return pl.kernel(
        _scat_body,
        out_type=jax.ShapeDtypeStruct((NC, NP, D), jnp.float32),
        mesh=plsc.VectorSubcoreMesh(
            core_axis_name="c", subcore_axis_name="s",
            num_cores=NC, num_subcores=NS,
        ),
        scratch_types=[
            pltpu.VMEM((SCH,), jnp.int32),
            pltpu.VMEM((SCH,), jnp.int32),
            pltpu.VMEM((SCH,), jnp.int32),
            pltpu.VMEM((SCH,), jnp.int32),
            pltpu.VMEM((SCH,), jnp.int32),
            pltpu.VMEM((SCH,), jnp.int32),
            pltpu.VMEM((SCH, D), jnp.float32),
            pltpu.VMEM((SCH, D), jnp.float32),
            pltpu.VMEM((SCH, D), jnp.float32),
            pltpu.VMEM_SHARED((NP, D), jnp.float32),
            pltpu.SemaphoreType.DMA,
            pltpu.SemaphoreType.DMA,
            pltpu.SemaphoreType.DMA,
        ],
    )


BR = 1000
NB = N // BR


def _k1_body(dp_ref, x_ref, w_ref, hp_ref, dinv_ref):
    deg = 1.0 + dp_ref[0, :, 0:1] + dp_ref[1, :, 0:1]
    dinv = lax.rsqrt(deg)
    h = jnp.dot(x_ref[...], w_ref[...], preferred_element_type=jnp.float32)
    hp_ref[...] = h * dinv
    dinv_ref[...] = dinv


_k1_call = pl.pallas_call(
    _k1_body,
    grid=(NB,),
    in_specs=[
        pl.BlockSpec((NC, BR, DW), lambda i: (0, i, 0)),
        pl.BlockSpec((BR, D), lambda i: (i, 0)),
        pl.BlockSpec((D, D), lambda i: (0, 0)),
    ],
    out_specs=[
        pl.BlockSpec((BR, D), lambda i: (i, 0)),
        pl.BlockSpec((BR, 1), lambda i: (i, 0)),
    ],
    out_shape=[
        jax.ShapeDtypeStruct((N, D), jnp.float32),
        jax.ShapeDtypeStruct((N, 1), jnp.float32),
    ],
)


def _acc_body(p_ref, hp_ref, dinv_ref, b_ref, y_ref, st_ref, acc):
    i = pl.program_id(0)

    @pl.when(i == 0)
    def _():
        acc[...] = jnp.zeros_like(acc)

    y = dinv_ref[...] * (p_ref[0] + p_ref[1] + hp_ref[...]) + b_ref[...]
    y_ref[...] = y
    acc[0:1, :] = acc[0:1, :] + jnp.sum(y, axis=0, keepdims=True)
    acc[1:2, :] = acc[1:2, :] + jnp.sum(y * y, axis=0, keepdims=True)
    st_ref[...] = acc[...]


_acc_call = pl.pallas_call(
    _acc_body,
    grid=(NB,),
    in_specs=[
        pl.BlockSpec((NC, BR, D), lambda i: (0, i, 0)),
        pl.BlockSpec((BR, D), lambda i: (i, 0)),
        pl.BlockSpec((BR, 1), lambda i: (i, 0)),
        pl.BlockSpec((1, D), lambda i: (0, 0)),
    ],
    out_specs=[
        pl.BlockSpec((BR, D), lambda i: (i, 0)),
        pl.BlockSpec((2, D), lambda i: (0, 0)),
    ],
    out_shape=[
        jax.ShapeDtypeStruct((N, D), jnp.float32),
        jax.ShapeDtypeStruct((2, D), jnp.float32),
    ],
    scratch_shapes=[pltpu.VMEM((2, D), jnp.float32)],
)


def _bn_prelu(y, st, g_ref, be_ref, a_ref):
    mu = st[0:1, :] * (1.0 / N)
    var = st[1:2, :] * (1.0 / N) - mu * mu
    z = (y - mu) * lax.rsqrt(var + 1e-5) * g_ref[...] + be_ref[...]
    return jnp.where(z > 0, z, a_ref[0, 0] * z)


def _mid_body(y_ref, st_ref, g_ref, be_ref, a_ref, dinv_ref, w_ref, out_ref):
    p = _bn_prelu(y_ref[...], st_ref[...], g_ref, be_ref, a_ref)
    h = jnp.dot(p, w_ref[...], preferred_element_type=jnp.float32)
    out_ref[...] = h * dinv_ref[...]


_mid_call = pl.pallas_call(
    _mid_body,
    grid=(NB,),
    in_specs=[
        pl.BlockSpec((BR, D), lambda i: (i, 0)),
        pl.BlockSpec((2, D), lambda i: (0, 0)),
        pl.BlockSpec((1, D), lambda i: (0, 0)),
        pl.BlockSpec((1, D), lambda i: (0, 0)),
        pl.BlockSpec((1, 1), lambda i: (0, 0)),
        pl.BlockSpec((BR, 1), lambda i: (i, 0)),
        pl.BlockSpec((D, D), lambda i: (0, 0)),
    ],
    out_specs=pl.BlockSpec((BR, D), lambda i: (i, 0)),
    out_shape=jax.ShapeDtypeStruct((N, D), jnp.float32),
)


def _fin_body(y_ref, st_ref, g_ref, be_ref, a_ref, out_ref):
    out_ref[...] = _bn_prelu(y_ref[...], st_ref[...], g_ref, be_ref, a_ref)


_fin_call = pl.pallas_call(
    _fin_body,
    grid=(NB,),
    in_specs=[
        pl.BlockSpec((BR, D), lambda i: (i, 0)),
        pl.BlockSpec((2, D), lambda i: (0, 0)),
        pl.BlockSpec((1, D), lambda i: (0, 0)),
        pl.BlockSpec((1, D), lambda i: (0, 0)),
        pl.BlockSpec((1, 1), lambda i: (0, 0)),
    ],
    out_specs=pl.BlockSpec((BR, D), lambda i: (i, 0)),
    out_shape=jax.ShapeDtypeStruct((N, D), jnp.float32),
)


def kernel(x, edge_index, W1, b1, g1, be1, a1, W2, b2, g2, be2, a2):
    src = edge_index[0]
    dst = edge_index[1]
    b1r, b2r = b1.reshape(1, D), b2.reshape(1, D)
    g1r, g2r = g1.reshape(1, D), g2.reshape(1, D)
    be1r, be2r = be1.reshape(1, D), be2.reshape(1, D)
    a1r, a2r = a1.reshape(1, 1), a2.reshape(1, 1)

    degp = _deg_call()(dst)
    hp1, dinv = _k1_call(degp, x, W1)
    p1 = _scat_call()(hp1, src, dst)
    y1, st1 = _acc_call(p1, hp1, dinv, b1r)
    hp2 = _mid_call(y1, st1, g1r, be1r, a1r, dinv, W2)
    p2 = _scat_call()(hp2, src, dst)
    y2, st2 = _acc_call(p2, hp2, dinv, b2r)
    return _fin_call(y2, st2, g2r, be2r, a2r)


# speedup vs baseline: 1.1297x; 1.1254x over previous
import functools

import jax
import jax.numpy as jnp
from jax import lax
from jax.experimental import pallas as pl
from jax.experimental.pallas import tpu as pltpu
from jax.experimental.pallas import tpu_sc as plsc

N = 10000
E = 320000
D = 128

NC = 2
NS = 16
NW = NC * NS
EPW = E // NW
CHS = 128
NFC = EPW // CHS
TL = EPW - NFC * CHS
NP = 10240
RPS = NP // NS
ZR = 128
DW = 16
SCH = 80
SNC = EPW // SCH


DBL = 2000


def _deg_body(dst_hbm, out_hbm, idx_v, hist_v, tmp_v, sum_v, exp_v, hists_sh):
    c = lax.axis_index("c")
    s = lax.axis_index("s")
    wid = c * NS + s

    zero16 = jnp.zeros((16,), jnp.float32)
    ones16 = jnp.ones((16,), jnp.float32)

    @pl.loop(0, NP // 16)
    def _zh(i):
        hist_v[pl.ds(i * 16, 16)] = zero16

    for b in range(EPW // DBL):
        pltpu.sync_copy(dst_hbm.at[pl.ds(wid * EPW + b * DBL, DBL)], idx_v)

        @pl.loop(0, DBL // (16 * 5))
        def _hist(j):
            for u in range(5):
                iv = idx_v[pl.ds((j * 5 + u) * 16, 16)]
                plsc.addupdate_scatter(hist_v, [iv], ones16)

    pltpu.sync_copy(hist_v, hists_sh.at[s])
    plsc.subcore_barrier()

    pltpu.sync_copy(hists_sh.at[:, pl.ds(s * RPS, RPS)], tmp_v)

    @pl.loop(0, RPS // 16)
    def _sum(g):
        acc = tmp_v[0, pl.ds(g * 16, 16)]
        for t in range(1, NS):
            acc = acc + tmp_v[t, pl.ds(g * 16, 16)]
        sum_v[pl.ds(g * 16, 16)] = acc

    @pl.loop(0, RPS // 16)
    def _exp(g):
        base = g * 16
        for r in range(16):
            iv = base + r + jnp.zeros((16,), jnp.int32)
            exp_v[base + r] = plsc.load_gather(sum_v, [iv])

    pltpu.sync_copy(exp_v, out_hbm.at[c, pl.ds(s * RPS, RPS)])


@functools.cache
def _deg_call():
    return pl.kernel(
        _deg_body,
        out_type=jax.ShapeDtypeStruct((NC, NP, DW), jnp.float32),
        mesh=plsc.VectorSubcoreMesh(
            core_axis_name="c", subcore_axis_name="s",
            num_cores=NC, num_subcores=NS,
        ),
        compiler_params=pltpu.CompilerParams(needs_layout_passes=False),
        scratch_types=[
            pltpu.VMEM((DBL,), jnp.int32),
            pltpu.VMEM((NP,), jnp.float32),
            pltpu.VMEM((NS, RPS), jnp.float32),
            pltpu.VMEM((RPS,), jnp.float32),
            pltpu.VMEM((RPS, DW), jnp.float32),
            pltpu.VMEM_SHARED((NS, NP), jnp.float32),
        ],
    )


def _scat_body(hp_hbm, src_hbm, dst_hbm, out_hbm,
               idxs0_v, idxd0_v, idxs1_v, idxd1_v, idxs2_v, idxd2_v,
               rows0_v, rows1_v, rows2_v, acc_sh, sem0, sem1, sem2):
    c = lax.axis_index("c")
    s = lax.axis_index("s")
    wid = c * NS + s

    idxs = (idxs0_v, idxs1_v, idxs2_v)
    idxd = (idxd0_v, idxd1_v, idxd2_v)
    rows = (rows0_v, rows1_v, rows2_v)
    sems = (sem0, sem1, sem2)

    @pl.loop(0, SCH)
    def _zfill(i):
        for j in range(D // 16):
            rows0_v[i, pl.ds(j * 16, 16)] = jnp.zeros((16,), jnp.float32)

    @pl.loop(0, RPS // SCH)
    def _zero(k):
        pltpu.sync_copy(rows0_v, acc_sh.at[pl.ds(s * RPS + k * SCH, SCH)])

    plsc.subcore_barrier()

    def _issue(i, k):
        e0 = wid * EPW + i * SCH
        pltpu.sync_copy(src_hbm.at[pl.ds(e0, SCH)], idxs[k])
        pltpu.sync_copy(dst_hbm.at[pl.ds(e0, SCH)], idxd[k])
        pltpu.async_copy(hp_hbm.at[idxs[k]], rows[k], sems[k])

    def _drain(k):
        pltpu.make_async_copy(hp_hbm.at[idxs[k]], rows[k], sems[k]).wait()
        pltpu.sync_copy(rows[k], acc_sh.at[idxd[k]], add=True)

    for k in range(3):
        _issue(k, k)

    @pl.loop(0, SNC - 5, step=3)
    def _edges(i):
        for k in range(3):
            _drain(k)
            _issue(i + 3 + k, k)

    _drain(0)
    _issue(SNC - 2, 0)
    _drain(1)
    _issue(SNC - 1, 1)
    _drain(2)
    _drain(0)
    _drain(1)

    plsc.subcore_barrier()
    pltpu.sync_copy(
        acc_sh.at[pl.ds(s * RPS, RPS)], out_hbm.at[c, pl.ds(s * RPS, RPS)]
    )


@functools.cache
def _scat_call():
    return pl.kernel(
        _scat_body,
        out_type=jax.ShapeDtypeStruct((NC, NP, D), jnp.float32),
        mesh=plsc.VectorSubcoreMesh(
            core_axis_name="c", subcore_axis_name="s",
            num_cores=NC, num_subcores=NS,
        ),
        scratch_types=[
            pltpu.VMEM((SCH,), jnp.int32),
            pltpu.VMEM((SCH,), jnp.int32),
            pltpu.VMEM((SCH,), jnp.int32),
            pltpu.VMEM((SCH,), jnp.int32),
            pltpu.VMEM((SCH,), jnp.int32),
            pltpu.VMEM((SCH,), jnp.int32),
            pltpu.VMEM((SCH, D), jnp.float32),
            pltpu.VMEM((SCH, D), jnp.float32),
            pltpu.VMEM((SCH, D), jnp.float32),
            pltpu.VMEM_SHARED((NP, D), jnp.float32),
            pltpu.SemaphoreType.DMA,
            pltpu.SemaphoreType.DMA,
            pltpu.SemaphoreType.DMA,
        ],
    )


BR = 1000
NB = N // BR


def _k1_body(dp_ref, x_ref, w_ref, hp_ref, dinv_ref):
    deg = 1.0 + dp_ref[0, :, 0:1] + dp_ref[1, :, 0:1]
    dinv = lax.rsqrt(deg)
    h = jnp.dot(x_ref[...], w_ref[...], preferred_element_type=jnp.float32)
    hp_ref[...] = h * dinv
    dinv_ref[...] = dinv


_k1_call = pl.pallas_call(
    _k1_body,
    grid=(NB,),
    in_specs=[
        pl.BlockSpec((NC, BR, DW), lambda i: (0, i, 0)),
        pl.BlockSpec((BR, D), lambda i: (i, 0)),
        pl.BlockSpec((D, D), lambda i: (0, 0)),
    ],
    out_specs=[
        pl.BlockSpec((BR, D), lambda i: (i, 0)),
        pl.BlockSpec((BR, 1), lambda i: (i, 0)),
    ],
    out_shape=[
        jax.ShapeDtypeStruct((N, D), jnp.float32),
        jax.ShapeDtypeStruct((N, 1), jnp.float32),
    ],
)


def _acc_body(p_ref, hp_ref, dinv_ref, b_ref, y_ref, st_ref, acc):
    i = pl.program_id(0)

    @pl.when(i == 0)
    def _():
        acc[...] = jnp.zeros_like(acc)

    y = dinv_ref[...] * (p_ref[0] + p_ref[1] + hp_ref[...]) + b_ref[...]
    y_ref[...] = y
    acc[0:1, :] = acc[0:1, :] + jnp.sum(y, axis=0, keepdims=True)
    acc[1:2, :] = acc[1:2, :] + jnp.sum(y * y, axis=0, keepdims=True)
    st_ref[...] = acc[...]


_acc_call = pl.pallas_call(
    _acc_body,
    grid=(NB,),
    in_specs=[
        pl.BlockSpec((NC, BR, D), lambda i: (0, i, 0)),
        pl.BlockSpec((BR, D), lambda i: (i, 0)),
        pl.BlockSpec((BR, 1), lambda i: (i, 0)),
        pl.BlockSpec((1, D), lambda i: (0, 0)),
    ],
    out_specs=[
        pl.BlockSpec((BR, D), lambda i: (i, 0)),
        pl.BlockSpec((2, D), lambda i: (0, 0)),
    ],
    out_shape=[
        jax.ShapeDtypeStruct((N, D), jnp.float32),
        jax.ShapeDtypeStruct((2, D), jnp.float32),
    ],
    scratch_shapes=[pltpu.VMEM((2, D), jnp.float32)],
)


def _bn_prelu(y, st, g_ref, be_ref, a_ref):
    mu = st[0:1, :] * (1.0 / N)
    var = st[1:2, :] * (1.0 / N) - mu * mu
    z = (y - mu) * lax.rsqrt(var + 1e-5) * g_ref[...] + be_ref[...]
    return jnp.where(z > 0, z, a_ref[0, 0] * z)


def _mid_body(y_ref, st_ref, g_ref, be_ref, a_ref, dinv_ref, w_ref, out_ref):
    p = _bn_prelu(y_ref[...], st_ref[...], g_ref, be_ref, a_ref)
    h = jnp.dot(p, w_ref[...], preferred_element_type=jnp.float32)
    out_ref[...] = h * dinv_ref[...]


_mid_call = pl.pallas_call(
    _mid_body,
    grid=(NB,),
    in_specs=[
        pl.BlockSpec((BR, D), lambda i: (i, 0)),
        pl.BlockSpec((2, D), lambda i: (0, 0)),
        pl.BlockSpec((1, D), lambda i: (0, 0)),
        pl.BlockSpec((1, D), lambda i: (0, 0)),
        pl.BlockSpec((1, 1), lambda i: (0, 0)),
        pl.BlockSpec((BR, 1), lambda i: (i, 0)),
        pl.BlockSpec((D, D), lambda i: (0, 0)),
    ],
    out_specs=pl.BlockSpec((BR, D), lambda i: (i, 0)),
    out_shape=jax.ShapeDtypeStruct((N, D), jnp.float32),
)


def _fin_body(y_ref, st_ref, g_ref, be_ref, a_ref, out_ref):
    out_ref[...] = _bn_prelu(y_ref[...], st_ref[...], g_ref, be_ref, a_ref)


_fin_call = pl.pallas_call(
    _fin_body,
    grid=(NB,),
    in_specs=[
        pl.BlockSpec((BR, D), lambda i: (i, 0)),
        pl.BlockSpec((2, D), lambda i: (0, 0)),
        pl.BlockSpec((1, D), lambda i: (0, 0)),
        pl.BlockSpec((1, D), lambda i: (0, 0)),
        pl.BlockSpec((1, 1), lambda i: (0, 0)),
    ],
    out_specs=pl.BlockSpec((BR, D), lambda i: (i, 0)),
    out_shape=jax.ShapeDtypeStruct((N, D), jnp.float32),
)


def kernel(x, edge_index, W1, b1, g1, be1, a1, W2, b2, g2, be2, a2):
    src = edge_index[0]
    dst = edge_index[1]
    b1r, b2r = b1.reshape(1, D), b2.reshape(1, D)
    g1r, g2r = g1.reshape(1, D), g2.reshape(1, D)
    be1r, be2r = be1.reshape(1, D), be2.reshape(1, D)
    a1r, a2r = a1.reshape(1, 1), a2.reshape(1, 1)

    degp = _deg_call()(dst)
    hp1, dinv = _k1_call(degp, x, W1)
    p1 = _scat_call()(hp1, src, dst)
    y1, st1 = _acc_call(p1, hp1, dinv, b1r)
    hp2 = _mid_call(y1, st1, g1r, be1r, a1r, dinv, W2)
    p2 = _scat_call()(hp2, src, dst)
    y2, st2 = _acc_call(p2, hp2, dinv, b2r)
    return _fin_call(y2, st2, g2r, be2r, a2r)
